# baseline (speedup 1.0000x reference)
import functools

import jax
import jax.numpy as jnp
from jax import lax
from jax.experimental import pallas as pl
from jax.experimental.pallas import tpu as pltpu
from jax.experimental.pallas import tpu_sc as plsc

V_MIN = -10.0
V_MAX = 10.0
NUM_ATOMS = 51
DISCOUNT = 0.99
ATOM_DELTA = (V_MAX - V_MIN) / (NUM_ATOMS - 1)
INV_DELTA = 2.5

NC = 2
NS = 16
NW = NC * NS
LANES = 16
CHUNK = 128
ROW = 128


def _sc_body(rows_w, rew_hbm, nd_hbm, probs_hbm, out_hbm,
             rew_v, nd_v, pc0, pc1, oc0, oc1,
             psem0, psem1, osem0, osem1, rsem):
    wid = lax.axis_index("s") * NC + lax.axis_index("c")
    base = wid * rows_w
    nch = rows_w // CHUNK
    cwords = CHUNK * ROW
    pbufs, obufs = [pc0, pc1], [oc0, oc1]
    psems, osems = [psem0, psem1], [osem0, osem1]

    iota128 = lax.iota(jnp.int32, LANES) * ROW
    zeros16 = jnp.zeros((LANES,), jnp.float32)
    nblocks = CHUNK // LANES

    rdma = pltpu.async_copy(rew_hbm.at[pl.ds(base, rows_w)], rew_v, rsem)
    ndma = pltpu.async_copy(nd_hbm.at[pl.ds(base, rows_w)], nd_v, rsem)
    pdma = [None] * nch
    odma = [None] * nch
    pdma[0] = pltpu.async_copy(
        probs_hbm.at[pl.ds(base * ROW, cwords)], pbufs[0], psems[0])
    rdma.wait()
    ndma.wait()

    for c in range(nch):
        if c + 1 < nch:
            pdma[c + 1] = pltpu.async_copy(
                probs_hbm.at[pl.ds((base + (c + 1) * CHUNK) * ROW, cwords)],
                pbufs[(c + 1) % 2], psems[(c + 1) % 2])
        pdma[c].wait()
        if c >= 2:
            odma[c - 2].wait()
        pbuf, obuf = pbufs[c % 2], obufs[c % 2]
        cb = c * CHUNK

        def block(b, _):
            rowv = iota128 + b * (LANES * ROW)
            rew = rew_v[pl.ds(cb + b * LANES, LANES)]
            g = nd_v[pl.ds(cb + b * LANES, LANES)] * DISCOUNT
            for k in range(NUM_ATOMS):
                plsc.store_scatter(obuf, [rowv + k], zeros16)
            for j in range(NUM_ATOMS):
                a_j = V_MIN + ATOM_DELTA * j
                p = plsc.load_gather(pbuf, [rowv + j])
                val = rew + g * a_j
                val = jnp.minimum(jnp.maximum(val, V_MIN), V_MAX)
                xf = (val - V_MIN) * INV_DELTA
                li = xf.astype(jnp.int32)
                frac = xf - li.astype(jnp.float32)
                uv = frac * p
                lv = p - uv
                sidx = rowv + li
                plsc.addupdate_scatter(obuf, [sidx], lv)
                plsc.addupdate_scatter(obuf, [sidx + 1], uv)
            return _

        lax.fori_loop(0, nblocks, block, None)
        odma[c] = pltpu.async_copy(
            obuf, out_hbm.at[pl.ds((base + c * CHUNK) * ROW, cwords)],
            osems[c % 2])

    odma[nch - 2].wait()
    odma[nch - 1].wait()


@jax.jit
def kernel(reward, probs, not_done):
    bs = probs.shape[0]
    rows_w = bs // NW
    probs_p = jnp.pad(probs, ((0, 0), (0, ROW - NUM_ATOMS))).reshape(-1)
    mesh = plsc.VectorSubcoreMesh(
        core_axis_name="c", subcore_axis_name="s",
        num_cores=NC, num_subcores=NS)
    run = pl.kernel(
        functools.partial(_sc_body, rows_w),
        out_type=jax.ShapeDtypeStruct((bs * ROW,), jnp.float32),
        mesh=mesh,
        compiler_params=pltpu.CompilerParams(needs_layout_passes=False),
        scratch_types=[
            pltpu.VMEM((rows_w,), jnp.float32),
            pltpu.VMEM((rows_w,), jnp.float32),
            pltpu.VMEM((CHUNK * ROW,), jnp.float32),
            pltpu.VMEM((CHUNK * ROW,), jnp.float32),
            pltpu.VMEM((CHUNK * ROW,), jnp.float32),
            pltpu.VMEM((CHUNK * ROW,), jnp.float32),
            pltpu.SemaphoreType.DMA,
            pltpu.SemaphoreType.DMA,
            pltpu.SemaphoreType.DMA,
            pltpu.SemaphoreType.DMA,
            pltpu.SemaphoreType.DMA,
        ],
    )
    out_p = run(reward.reshape(-1), not_done.reshape(-1), probs_p)
    return out_p.reshape(bs, ROW)[:, :NUM_ATOMS]


# speedup vs baseline: 1.3032x; 1.3032x over previous
import functools

import jax
import jax.numpy as jnp
from jax import lax
from jax.experimental import pallas as pl
from jax.experimental.pallas import tpu as pltpu
from jax.experimental.pallas import tpu_sc as plsc

V_MIN = -10.0
V_MAX = 10.0
NUM_ATOMS = 51
DISCOUNT = 0.99
ATOM_DELTA = (V_MAX - V_MIN) / (NUM_ATOMS - 1)
INV_DELTA = 2.5

NC = 2
NS = 16
NW = NC * NS
LANES = 16
CHUNK = 128
ROW = 128


def _sc_body(rows_w, rew_hbm, nd_hbm, probs_hbm, out_hbm,
             rew_v, nd_v, pc0, pc1, oc0, oc1,
             psem0, psem1, osem0, osem1, rsem):
    wid = lax.axis_index("s") * NC + lax.axis_index("c")
    base = wid * rows_w
    nch = rows_w // CHUNK
    cwords = CHUNK * ROW
    pbufs, obufs = [pc0, pc1], [oc0, oc1]
    psems, osems = [psem0, psem1], [osem0, osem1]

    iota128 = lax.iota(jnp.int32, LANES) * ROW
    zeros16 = jnp.zeros((LANES,), jnp.float32)
    nblocks = CHUNK // LANES
    ii = lax.iota(jnp.int32, LANES)
    colv, av = [], []
    for j in range(NUM_ATOMS):
        cj = ii + j
        cj = jnp.where(cj >= NUM_ATOMS, cj - NUM_ATOMS, cj)
        colv.append(cj)
        av.append(V_MIN + ATOM_DELTA * cj.astype(jnp.float32))

    rdma = pltpu.async_copy(rew_hbm.at[pl.ds(base, rows_w)], rew_v, rsem)
    ndma = pltpu.async_copy(nd_hbm.at[pl.ds(base, rows_w)], nd_v, rsem)
    pdma = [None] * nch
    odma = [None] * nch
    pdma[0] = pltpu.async_copy(
        probs_hbm.at[pl.ds(base * ROW, cwords)], pbufs[0], psems[0])
    rdma.wait()
    ndma.wait()

    for c in range(nch):
        if c + 1 < nch:
            pdma[c + 1] = pltpu.async_copy(
                probs_hbm.at[pl.ds((base + (c + 1) * CHUNK) * ROW, cwords)],
                pbufs[(c + 1) % 2], psems[(c + 1) % 2])
        pdma[c].wait()
        if c >= 2:
            odma[c - 2].wait()
        pbuf, obuf = pbufs[c % 2], obufs[c % 2]
        cb = c * CHUNK

        def zero(i, _):
            off = i * (8 * LANES)
            for k in range(8):
                obuf[pl.ds(off + k * LANES, LANES)] = zeros16
            return _

        lax.fori_loop(0, cwords // (8 * LANES), zero, None)

        def block(b, _):
            rowv = iota128 + b * (LANES * ROW)
            rew = rew_v[pl.ds(cb + b * LANES, LANES)]
            g = nd_v[pl.ds(cb + b * LANES, LANES)] * DISCOUNT
            for j in range(NUM_ATOMS):
                p = plsc.load_gather(pbuf, [rowv + colv[j]])
                val = rew + g * av[j]
                val = jnp.minimum(jnp.maximum(val, V_MIN), V_MAX)
                xf = (val - V_MIN) * INV_DELTA
                li = xf.astype(jnp.int32)
                frac = xf - li.astype(jnp.float32)
                uv = frac * p
                lv = p - uv
                sidx = rowv + li
                plsc.addupdate_scatter(obuf, [sidx], lv)
                plsc.addupdate_scatter(obuf, [sidx + 1], uv)
            return _

        lax.fori_loop(0, nblocks, block, None)
        odma[c] = pltpu.async_copy(
            obuf, out_hbm.at[pl.ds((base + c * CHUNK) * ROW, cwords)],
            osems[c % 2])

    odma[nch - 2].wait()
    odma[nch - 1].wait()


@jax.jit
def kernel(reward, probs, not_done):
    bs = probs.shape[0]
    rows_w = bs // NW
    probs_p = jnp.pad(probs, ((0, 0), (0, ROW - NUM_ATOMS))).reshape(-1)
    mesh = plsc.VectorSubcoreMesh(
        core_axis_name="c", subcore_axis_name="s",
        num_cores=NC, num_subcores=NS)
    run = pl.kernel(
        functools.partial(_sc_body, rows_w),
        out_type=jax.ShapeDtypeStruct((bs * ROW,), jnp.float32),
        mesh=mesh,
        compiler_params=pltpu.CompilerParams(needs_layout_passes=False),
        scratch_types=[
            pltpu.VMEM((rows_w,), jnp.float32),
            pltpu.VMEM((rows_w,), jnp.float32),
            pltpu.VMEM((CHUNK * ROW,), jnp.float32),
            pltpu.VMEM((CHUNK * ROW,), jnp.float32),
            pltpu.VMEM((CHUNK * ROW,), jnp.float32),
            pltpu.VMEM((CHUNK * ROW,), jnp.float32),
            pltpu.SemaphoreType.DMA,
            pltpu.SemaphoreType.DMA,
            pltpu.SemaphoreType.DMA,
            pltpu.SemaphoreType.DMA,
            pltpu.SemaphoreType.DMA,
        ],
    )
    out_p = run(reward.reshape(-1), not_done.reshape(-1), probs_p)
    return out_p.reshape(bs, ROW)[:, :NUM_ATOMS]


# speedup vs baseline: 1.3263x; 1.0178x over previous
import functools

import jax
import jax.numpy as jnp
from jax import lax
from jax.experimental import pallas as pl
from jax.experimental.pallas import tpu as pltpu
from jax.experimental.pallas import tpu_sc as plsc

V_MIN = -10.0
V_MAX = 10.0
NUM_ATOMS = 51
DISCOUNT = 0.99
ATOM_DELTA = (V_MAX - V_MIN) / (NUM_ATOMS - 1)
INV_DELTA = 2.5

NC = 2
NS = 16
NW = NC * NS
LANES = 16
CHUNK = 128
ROW = 128


def _sc_body(rows_w, rew_hbm, nd_hbm, probs_hbm, out_hbm,
             rew_v, nd_v, pc0, pc1, oc0, oc1,
             psem0, psem1, osem0, osem1, rsem):
    wid = lax.axis_index("s") * NC + lax.axis_index("c")
    base = wid * rows_w
    nch = rows_w // CHUNK
    cwords = CHUNK * ROW
    pbufs, obufs = [pc0, pc1], [oc0, oc1]
    psems, osems = [psem0, psem1], [osem0, osem1]

    iota128 = lax.iota(jnp.int32, LANES) * ROW
    zeros16 = jnp.zeros((LANES,), jnp.float32)
    nblocks = CHUNK // LANES
    ii = lax.iota(jnp.int32, LANES)
    colv, av = [], []
    for j in range(NUM_ATOMS):
        cj = ii + j
        cj = jnp.where(cj >= NUM_ATOMS, cj - NUM_ATOMS, cj)
        colv.append(cj)
        av.append(V_MIN + ATOM_DELTA * cj.astype(jnp.float32))

    rdma = pltpu.async_copy(rew_hbm.at[pl.ds(base, rows_w)], rew_v, rsem)
    ndma = pltpu.async_copy(nd_hbm.at[pl.ds(base, rows_w)], nd_v, rsem)
    pdma = [None] * nch
    odma = [None] * nch
    pdma[0] = pltpu.async_copy(
        probs_hbm.at[pl.ds(base * ROW, cwords)], pbufs[0], psems[0])
    rdma.wait()
    ndma.wait()

    for c in range(nch):
        if c + 1 < nch:
            pdma[c + 1] = pltpu.async_copy(
                probs_hbm.at[pl.ds((base + (c + 1) * CHUNK) * ROW, cwords)],
                pbufs[(c + 1) % 2], psems[(c + 1) % 2])
        pdma[c].wait()
        if c >= 2:
            odma[c - 2].wait()
        pbuf, obuf = pbufs[c % 2], obufs[c % 2]
        cb = c * CHUNK

        def zero(i, _):
            off = i * (8 * LANES)
            for k in range(8):
                obuf[pl.ds(off + k * LANES, LANES)] = zeros16
            return _

        lax.fori_loop(0, cwords // (8 * LANES), zero, None)

        def block(b, _):
            rowv = iota128 + b * (LANES * ROW)
            rew = rew_v[pl.ds(cb + b * LANES, LANES)]
            nd = nd_v[pl.ds(cb + b * LANES, LANES)]
            A = rew * INV_DELTA - (V_MIN * INV_DELTA)
            B = nd * (DISCOUNT * INV_DELTA)
            for j in range(NUM_ATOMS):
                p = plsc.load_gather(pbuf, [rowv + colv[j]])
                xf = A + B * av[j]
                xf = jnp.minimum(jnp.maximum(xf, 0.0), 50.0)
                li = xf.astype(jnp.int32)
                frac = xf - li.astype(jnp.float32)
                uv = frac * p
                lv = p - uv
                sidx = rowv + li
                plsc.addupdate_scatter(obuf, [sidx], lv)
                plsc.addupdate_scatter(obuf, [sidx + 1], uv)
            return _

        lax.fori_loop(0, nblocks, block, None)
        odma[c] = pltpu.async_copy(
            obuf, out_hbm.at[pl.ds((base + c * CHUNK) * ROW, cwords)],
            osems[c % 2])

    odma[nch - 2].wait()
    odma[nch - 1].wait()


@jax.jit
def kernel(reward, probs, not_done):
    bs = probs.shape[0]
    rows_w = bs // NW
    probs_p = jnp.pad(probs, ((0, 0), (0, ROW - NUM_ATOMS))).reshape(-1)
    mesh = plsc.VectorSubcoreMesh(
        core_axis_name="c", subcore_axis_name="s",
        num_cores=NC, num_subcores=NS)
    run = pl.kernel(
        functools.partial(_sc_body, rows_w),
        out_type=jax.ShapeDtypeStruct((bs * ROW,), jnp.float32),
        mesh=mesh,
        compiler_params=pltpu.CompilerParams(needs_layout_passes=False),
        scratch_types=[
            pltpu.VMEM((rows_w,), jnp.float32),
            pltpu.VMEM((rows_w,), jnp.float32),
            pltpu.VMEM((CHUNK * ROW,), jnp.float32),
            pltpu.VMEM((CHUNK * ROW,), jnp.float32),
            pltpu.VMEM((CHUNK * ROW,), jnp.float32),
            pltpu.VMEM((CHUNK * ROW,), jnp.float32),
            pltpu.SemaphoreType.DMA,
            pltpu.SemaphoreType.DMA,
            pltpu.SemaphoreType.DMA,
            pltpu.SemaphoreType.DMA,
            pltpu.SemaphoreType.DMA,
        ],
    )
    out_p = run(reward.reshape(-1), not_done.reshape(-1), probs_p)
    return out_p.reshape(bs, ROW)[:, :NUM_ATOMS]
